# baseline (speedup 1.0000x reference)
import functools

import jax
import jax.numpy as jnp
from jax import lax
from jax.experimental import pallas as pl
from jax.experimental.pallas import tpu as pltpu
from jax.experimental.pallas import tpu_sc as plsc

N = 10000
E = 320000
F_IN = 128
HEADS = 8
C1 = 16
D1 = HEADS * C1
NC_OUT = 40
D2P = 48

NCORES = 2
NSUB = 16
NW = NCORES * NSUB
PW = E // NW

CH_A = 400
NCH_A = PW // CH_A
CH_A2 = 2000
NCH_A2 = PW // CH_A2
CH_B1 = 200
NCH_B1 = PW // CH_B1
CH_B2 = 400
NCH_B2 = PW // CH_B2

ROWS_T = N // NSUB

f32 = jnp.float32
i32 = jnp.int32

_MESH = plsc.VectorSubcoreMesh(core_axis_name="c", subcore_axis_name="s")


def _leaky(a):
    return jnp.where(a >= 0.0, a, 0.2 * a)


def _pre1_body(x_ref, w_ref, bs_ref, bd_ref, h_ref, as_ref, ad_ref, sh_ref):
    h = jnp.dot(x_ref[...], w_ref[...], preferred_element_type=f32)
    h_ref[...] = h
    a_s = jnp.dot(h, bs_ref[...], preferred_element_type=f32)
    a_d = jnp.dot(h, bd_ref[...], preferred_element_type=f32)
    as_ref[...] = a_s
    ad_ref[...] = a_d
    m = jnp.max(a_s, axis=0, keepdims=True) + jnp.max(a_d, axis=0, keepdims=True)
    sh_ref[...] = _leaky(m)


def _pre1(x, W1, Bs16, Bd16):
    return pl.pallas_call(
        _pre1_body,
        out_shape=[
            jax.ShapeDtypeStruct((N, D1), f32),
            jax.ShapeDtypeStruct((N, 16), f32),
            jax.ShapeDtypeStruct((N, 16), f32),
            jax.ShapeDtypeStruct((1, 16), f32),
        ],
    )(x, W1, Bs16, Bd16)


def _rdenom1_body(d_ref, rd_ref):
    dsum = jnp.sum(d_ref[...], axis=0, keepdims=True)
    rd_ref[...] = 1.0 / (dsum + 1e-16)


def _rdenom1(dparts):
    return pl.pallas_call(
        _rdenom1_body,
        out_shape=jax.ShapeDtypeStruct((1, N * HEADS), f32),
    )(dparts)


def _mid_body(p_ref, b1_ref, w2_ref, a2_ref, rd_ref, ex_ref,
              h2_ref, as2_ref, ad2_ref, sh2_ref):
    rdexp = jnp.dot(rd_ref[...], ex_ref[...], preferred_element_type=f32)
    o1 = (p_ref[0] + p_ref[1]) * rdexp + b1_ref[...]
    x2 = jnp.where(o1 > 0.0, o1, jnp.exp(jnp.minimum(o1, 0.0)) - 1.0)
    h2 = jnp.dot(x2, w2_ref[...], preferred_element_type=f32)
    h2_ref[...] = h2
    a2 = jnp.dot(h2, a2_ref[...], preferred_element_type=f32)
    a_s = a2[:, 0:1]
    a_d = a2[:, 1:2]
    as2_ref[...] = a_s
    ad2_ref[...] = a_d
    m = jnp.max(a_s) + jnp.max(a_d)
    sh2_ref[...] = _leaky(jnp.full((1, 16), m, dtype=f32))


def _mid(parts, bias1, W2p, A2, rd8, expand):
    return pl.pallas_call(
        _mid_body,
        out_shape=[
            jax.ShapeDtypeStruct((N, D2P), f32),
            jax.ShapeDtypeStruct((N, 1), f32),
            jax.ShapeDtypeStruct((N, 1), f32),
            jax.ShapeDtypeStruct((1, 16), f32),
        ],
    )(parts, bias1.reshape(1, D1), W2p, A2, rd8, expand)


def _rdenom2_body(d_ref, rd_ref):
    dsum = jnp.sum(d_ref[...], axis=0, keepdims=True)
    rd_ref[...] = 1.0 / (dsum + 1e-16)


def _rdenom2(dparts):
    return pl.pallas_call(
        _rdenom2_body,
        out_shape=jax.ShapeDtypeStruct((1, N), f32),
    )(dparts)


def _fin_body(p_ref, b2_ref, rd_ref, o_ref):
    o = (p_ref[0] + p_ref[1]) * rd_ref[...]
    o_ref[...] = o[:, :NC_OUT] + b2_ref[...]


def _fin(parts, bias2, rd2):
    return pl.pallas_call(
        _fin_body,
        out_shape=jax.ShapeDtypeStruct((N, NC_OUT), f32),
    )(parts, bias2.reshape(1, NC_OUT), rd2)


@functools.partial(
    pl.kernel,
    mesh=_MESH,
    compiler_params=pltpu.CompilerParams(needs_layout_passes=False, use_tc_tiling_on_sc=False),
    out_type=[
        jax.ShapeDtypeStruct((E * HEADS,), f32),
        jax.ShapeDtypeStruct((NW, N * HEADS), f32),
    ],
    scratch_types=[
        pltpu.VMEM((2, CH_A), i32),
        pltpu.VMEM((2, CH_A), i32),
        pltpu.VMEM((2, CH_A, 16), f32),
        pltpu.VMEM((2, CH_A, 16), f32),
        pltpu.VMEM((CH_A * HEADS,), f32),
        pltpu.VMEM((N * HEADS,), f32),
        pltpu.VMEM((16,), f32),
        pltpu.SemaphoreType.DMA,
        pltpu.SemaphoreType.DMA,
    ],
)
def _attn1(src_h, dst_h, as_h, ad_h, sh_h, ex_h, dp_h,
           srcb, dstb, g1, g2, exo, dacc, shv, sem0, sem1):
    wid = lax.axis_index("s") * NCORES + lax.axis_index("c")
    base = wid * PW
    sems = (sem0, sem1)
    pltpu.sync_copy(sh_h, shv)
    shift = shv[...]

    zero16 = jnp.zeros((16,), f32)

    def _z(i, c):
        dacc[pl.ds(i * 16, 16)] = zero16
        return c

    lax.fori_loop(0, N * HEADS // 16, _z, 0)

    def _start(c, b):
        off = base + c * CH_A
        pltpu.sync_copy(src_h.at[pl.ds(off, CH_A)], srcb.at[b])
        pltpu.sync_copy(dst_h.at[pl.ds(off, CH_A)], dstb.at[b])
        pltpu.async_copy(as_h.at[srcb.at[b]], g1.at[b], sems[b])
        pltpu.async_copy(ad_h.at[dstb.at[b]], g2.at[b], sems[b])

    def _finish(c, b):
        off = base + c * CH_A
        pltpu.make_async_copy(as_h.at[srcb.at[b]], g1.at[b], sems[b]).wait()
        pltpu.make_async_copy(ad_h.at[dstb.at[b]], g2.at[b], sems[b]).wait()

        def _group(g, cc):
            lane = jax.lax.broadcasted_iota(i32, (16,), 0)
            off8 = lane & 7
            hi = lane >> 3
            dvec = dstb[b, pl.ds(g * 16, 16)]
            for p in range(8):
                e0 = g * 16 + 2 * p
                e1 = e0 + 1
                a = jnp.where(lane < 8,
                              g1[b, e0, :] + g2[b, e0, :],
                              g1[b, e1, :] + g2[b, e1, :])
                ex = jnp.exp(_leaky(a) - shift)
                exo[pl.ds(g * 128 + p * 16, 16)] = ex
                sel = 2 * p + hi
                dd = dvec.at[sel].get(mode="promise_in_bounds")
                plsc.addupdate_scatter(dacc, [dd * HEADS + off8], ex)
            return cc

        lax.fori_loop(0, CH_A // 16, _group, 0)
        pltpu.sync_copy(exo, ex_h.at[pl.ds(off * HEADS, CH_A * HEADS)])

    _start(0, 0)

    def _outer(g, carry):
        c0 = g * 2
        _start(c0 + 1, 1)
        _finish(c0, 0)

        @pl.when(c0 + 2 < NCH_A)
        def _s():
            _start(c0 + 2, 0)

        _finish(c0 + 1, 1)
        return carry

    lax.fori_loop(0, NCH_A // 2, _outer, 0)
    if NCH_A % 2 == 1:
        _finish(NCH_A - 1, 0)
    pltpu.sync_copy(dacc, dp_h.at[wid])


@functools.partial(
    pl.kernel,
    mesh=_MESH,
    compiler_params=pltpu.CompilerParams(needs_layout_passes=False, use_tc_tiling_on_sc=False),
    out_type=jax.ShapeDtypeStruct((NCORES, N, D1), f32),
    scratch_types=[
        pltpu.VMEM((96,), i32),
        pltpu.VMEM((104,), i32),
        pltpu.VMEM((96,), i32),
        pltpu.VMEM((104,), i32),
        pltpu.VMEM((CH_B1, D1), f32),
        pltpu.VMEM((CH_B1 * HEADS,), f32),
        pltpu.VMEM_SHARED((N, D1), f32),
        pltpu.SemaphoreType.DMA,
        pltpu.SemaphoreType.DMA,
        pltpu.SemaphoreType.DMA,
        pltpu.SemaphoreType.DMA,
    ],
)
def _msg1(src_h, dst_h, h1_h, ex_h, out_h,
          srcbA, srcbB, dstbA, dstbB, hb, exb, osp,
          semGA, semGB, semSA, semSB):
    cc = lax.axis_index("c")
    sid = lax.axis_index("s")
    wid = sid * NCORES + cc
    base = wid * PW

    zero16 = jnp.zeros((16,), f32)
    for j in range(8):
        def _zr(r, c, j=j):
            hb[r, pl.ds(j * 16, 16)] = zero16
            return c
        lax.fori_loop(0, 125, _zr, 0)
    for k in range(5):
        pltpu.sync_copy(hb.at[pl.ds(0, 125)],
                        osp.at[pl.ds(sid * ROWS_T + k * 125, 125)])
    plsc.subcore_barrier()

    def _mul(p, ccar):
        zl = jax.lax.broadcasted_iota(i32, (16,), 0) * 0
        cvec = exb[pl.ds(p * 16, 16)]
        for half in range(2):
            e = 2 * p + half
            for j in range(HEADS):
                cj = cvec.at[zl + half * 8 + j].get(mode="promise_in_bounds")
                hb[e, pl.ds(j * 16, 16)] = hb[e, pl.ds(j * 16, 16)] * cj
        return ccar

    def _chunk(c, carry):
        off = base + c * CH_B1
        pltpu.sync_copy(src_h.at[pl.ds(off, 96)], srcbA)
        pltpu.sync_copy(src_h.at[pl.ds(off + 96, 104)], srcbB)
        pltpu.sync_copy(dst_h.at[pl.ds(off, 96)], dstbA)
        pltpu.sync_copy(dst_h.at[pl.ds(off + 96, 104)], dstbB)
        cpA = pltpu.async_copy(h1_h.at[srcbA], hb.at[pl.ds(0, 96)], semGA)
        cpB = pltpu.async_copy(h1_h.at[srcbB], hb.at[pl.ds(96, 104)], semGB)
        pltpu.sync_copy(ex_h.at[pl.ds(off * HEADS, CH_B1 * HEADS)], exb)

        cpA.wait()
        lax.fori_loop(0, 48, _mul, 0)

        cpB.wait()
        lax.fori_loop(48, CH_B1 // 2, _mul, 0)
        pltpu.sync_copy(hb.at[pl.ds(0, 96)], osp.at[dstbA], add=True)
        pltpu.sync_copy(hb.at[pl.ds(96, 104)], osp.at[dstbB], add=True)
        return carry

    lax.fori_loop(0, NCH_B1, _chunk, 0)
    plsc.subcore_barrier()

    @pl.when(sid == 0)
    def _copy_out():
        pltpu.sync_copy(osp, out_h.at[cc])


@functools.partial(
    pl.kernel,
    mesh=_MESH,
    compiler_params=pltpu.CompilerParams(needs_layout_passes=False, use_tc_tiling_on_sc=False),
    out_type=[
        jax.ShapeDtypeStruct((E,), f32),
        jax.ShapeDtypeStruct((NW, N), f32),
    ],
    scratch_types=[
        pltpu.VMEM((2, CH_A2), i32),
        pltpu.VMEM((2, CH_A2), i32),
        pltpu.VMEM((2, CH_A2), f32),
        pltpu.VMEM((2, CH_A2), f32),
        pltpu.VMEM((N,), f32),
        pltpu.VMEM((16,), f32),
        pltpu.SemaphoreType.DMA,
        pltpu.SemaphoreType.DMA,
    ],
)
def _attn2(src_h, dst_h, as_h, ad_h, sh_h, ex_h, dp_h,
           srcb, dstb, gs, gd, dacc, shv, sem0, sem1):
    wid = lax.axis_index("s") * NCORES + lax.axis_index("c")
    base = wid * PW
    sems = (sem0, sem1)
    pltpu.sync_copy(sh_h, shv)
    shift = shv[...]

    zero16 = jnp.zeros((16,), f32)

    def _z(i, c):
        dacc[pl.ds(i * 16, 16)] = zero16
        return c

    lax.fori_loop(0, N // 16, _z, 0)

    def _start(c, b):
        off = base + c * CH_A2
        pltpu.sync_copy(src_h.at[pl.ds(off, CH_A2)], srcb.at[b])
        pltpu.sync_copy(dst_h.at[pl.ds(off, CH_A2)], dstb.at[b])
        pltpu.async_copy(as_h.at[srcb.at[b]], gs.at[b], sems[b])
        pltpu.async_copy(ad_h.at[dstb.at[b]], gd.at[b], sems[b])

    def _finish(c, b):
        off = base + c * CH_A2
        pltpu.make_async_copy(as_h.at[srcb.at[b]], gs.at[b], sems[b]).wait()
        pltpu.make_async_copy(ad_h.at[dstb.at[b]], gd.at[b], sems[b]).wait()

        def _group(g, cg):
            a = _leaky(gs[b, pl.ds(g * 16, 16)] + gd[b, pl.ds(g * 16, 16)])
            ex = jnp.exp(a - shift)
            gs[b, pl.ds(g * 16, 16)] = ex
            dvec = dstb[b, pl.ds(g * 16, 16)]
            plsc.addupdate_scatter(dacc, [dvec], ex)
            return cg

        lax.fori_loop(0, CH_A2 // 16, _group, 0)
        pltpu.sync_copy(gs.at[b], ex_h.at[pl.ds(off, CH_A2)])

    _start(0, 0)

    def _outer(g, carry):
        c0 = g * 2
        _start(c0 + 1, 1)
        _finish(c0, 0)

        @pl.when(c0 + 2 < NCH_A2)
        def _s():
            _start(c0 + 2, 0)

        _finish(c0 + 1, 1)
        return carry

    lax.fori_loop(0, NCH_A2 // 2, _outer, 0)
    if NCH_A2 % 2 == 1:
        _finish(NCH_A2 - 1, 0)
    pltpu.sync_copy(dacc, dp_h.at[wid])


@functools.partial(
    pl.kernel,
    mesh=_MESH,
    compiler_params=pltpu.CompilerParams(needs_layout_passes=False, use_tc_tiling_on_sc=False),
    out_type=jax.ShapeDtypeStruct((NCORES, N, D2P), f32),
    scratch_types=[
        pltpu.VMEM((2, CH_B2), i32),
        pltpu.VMEM((2, CH_B2), i32),
        pltpu.VMEM((2, CH_B2, D2P), f32),
        pltpu.VMEM((2, CH_B2), f32),
        pltpu.VMEM_SHARED((N, D2P), f32),
        pltpu.SemaphoreType.DMA,
        pltpu.SemaphoreType.DMA,
    ],
)
def _msg2(src_h, dst_h, h2_h, ex_h, out_h,
          srcb, dstb, hb, exb, osp, sem0, sem1):
    cc = lax.axis_index("c")
    sid = lax.axis_index("s")
    wid = sid * NCORES + cc
    base = wid * PW
    sems = (sem0, sem1)

    zero16 = jnp.zeros((16,), f32)
    for j in range(D2P // 16):
        def _zr(r, c, j=j):
            hb[0, r, pl.ds(j * 16, 16)] = zero16
            return c
        lax.fori_loop(0, 125, _zr, 0)
    for k in range(5):
        pltpu.sync_copy(hb.at[0, pl.ds(0, 125)],
                        osp.at[pl.ds(sid * ROWS_T + k * 125, 125)])
    plsc.subcore_barrier()

    def _start(c, b):
        off = base + c * CH_B2
        pltpu.sync_copy(src_h.at[pl.ds(off, CH_B2)], srcb.at[b])
        pltpu.sync_copy(dst_h.at[pl.ds(off, CH_B2)], dstb.at[b])
        pltpu.sync_copy(ex_h.at[pl.ds(off, CH_B2)], exb.at[b])
        pltpu.async_copy(h2_h.at[srcb.at[b]], hb.at[b], sems[b])

    def _finish(b):
        pltpu.make_async_copy(h2_h.at[srcb.at[b]], hb.at[b], sems[b]).wait()

        def _group(g, cg):
            zl = jax.lax.broadcasted_iota(i32, (16,), 0) * 0
            cvec = exb[b, pl.ds(g * 16, 16)]
            for k in range(16):
                e = g * 16 + k
                ck = cvec.at[zl + k].get(mode="promise_in_bounds")
                for j in range(D2P // 16):
                    hb[b, e, pl.ds(j * 16, 16)] = hb[b, e, pl.ds(j * 16, 16)] * ck
            return cg

        lax.fori_loop(0, CH_B2 // 16, _group, 0)
        pltpu.sync_copy(hb.at[b], osp.at[dstb.at[b]], add=True)

    _start(0, 0)

    def _outer(g, carry):
        c0 = g * 2
        _start(c0 + 1, 1)
        _finish(0)

        @pl.when(c0 + 2 < NCH_B2)
        def _s():
            _start(c0 + 2, 0)

        _finish(1)
        return carry

    lax.fori_loop(0, NCH_B2 // 2, _outer, 0)
    if NCH_B2 % 2 == 1:
        _finish(0)
    plsc.subcore_barrier()

    @pl.when(sid == 0)
    def _copy_out():
        pltpu.sync_copy(osp, out_h.at[cc])


def kernel(x, edge_index, W1, att_src1, att_dst1, bias1, W2, att_src2, att_dst2, bias2):
    src = edge_index[0]
    dst = edge_index[1]

    rows = jnp.arange(D1)
    cols = rows // C1
    Bs8 = jnp.zeros((D1, HEADS), f32).at[rows, cols].set(att_src1.reshape(D1))
    Bd8 = jnp.zeros((D1, HEADS), f32).at[rows, cols].set(att_dst1.reshape(D1))
    Bs16 = jnp.concatenate([Bs8, Bs8], axis=1)
    Bd16 = jnp.concatenate([Bd8, Bd8], axis=1)

    h1, as16, ad16, sh1 = _pre1(x, W1, Bs16, Bd16)

    ex1, dparts1 = _attn1(src, dst, as16, ad16, sh1.reshape(16))
    rd1 = _rdenom1(dparts1)
    rd8 = rd1.reshape(N, HEADS)
    expand = jnp.repeat(jnp.eye(HEADS, dtype=f32), C1, axis=1)

    out1_parts = _msg1(src, dst, h1, ex1)

    W2p = jnp.concatenate([W2, jnp.zeros((D1, D2P - NC_OUT), f32)], axis=1)
    a2s = jnp.concatenate([att_src2.reshape(NC_OUT, 1),
                           jnp.zeros((D2P - NC_OUT, 1), f32)], axis=0)
    a2d = jnp.concatenate([att_dst2.reshape(NC_OUT, 1),
                           jnp.zeros((D2P - NC_OUT, 1), f32)], axis=0)
    A2 = jnp.concatenate([a2s, a2d], axis=1)

    h2, as2, ad2, sh2 = _mid(out1_parts, bias1, W2p, A2, rd8, expand)

    ex2, dparts2 = _attn2(src, dst, as2.reshape(N), ad2.reshape(N), sh2.reshape(16))
    rd2 = _rdenom2(dparts2).reshape(N, 1)

    out2_parts = _msg2(src, dst, h2, ex2)

    return _fin(out2_parts, bias2, rd2)


# speedup vs baseline: 1.0629x; 1.0629x over previous
import functools

import jax
import jax.numpy as jnp
from jax import lax
from jax.experimental import pallas as pl
from jax.experimental.pallas import tpu as pltpu
from jax.experimental.pallas import tpu_sc as plsc

N = 10000
E = 320000
F_IN = 128
HEADS = 8
C1 = 16
D1 = HEADS * C1
NC_OUT = 40
D2P = 48

NCORES = 2
NSUB = 16
NW = NCORES * NSUB
PW = E // NW

CH_A = 400
NCH_A = PW // CH_A
CH_A2 = 2000
NCH_A2 = PW // CH_A2
CH_B1 = 200
NCH_B1 = PW // CH_B1
CH_B2 = 400
NCH_B2 = PW // CH_B2

ROWS_T = N // NSUB

f32 = jnp.float32
i32 = jnp.int32

_MESH = plsc.VectorSubcoreMesh(core_axis_name="c", subcore_axis_name="s")


def _leaky(a):
    return jnp.where(a >= 0.0, a, 0.2 * a)


def _pre1_body(x_ref, w_ref, bs_ref, bd_ref, h_ref, as_ref, ad_ref, sh_ref):
    h = jnp.dot(x_ref[...], w_ref[...], preferred_element_type=f32)
    h_ref[...] = h
    a_s = jnp.dot(h, bs_ref[...], preferred_element_type=f32)
    a_d = jnp.dot(h, bd_ref[...], preferred_element_type=f32)
    as_ref[...] = a_s
    ad_ref[...] = a_d
    m = jnp.max(a_s, axis=0, keepdims=True) + jnp.max(a_d, axis=0, keepdims=True)
    sh_ref[...] = _leaky(m)


def _pre1(x, W1, Bs16, Bd16):
    return pl.pallas_call(
        _pre1_body,
        out_shape=[
            jax.ShapeDtypeStruct((N, D1), f32),
            jax.ShapeDtypeStruct((N, 16), f32),
            jax.ShapeDtypeStruct((N, 16), f32),
            jax.ShapeDtypeStruct((1, 16), f32),
        ],
    )(x, W1, Bs16, Bd16)


def _rdenom1_body(d_ref, rd_ref):
    dsum = jnp.sum(d_ref[...], axis=0, keepdims=True)
    rd_ref[...] = 1.0 / (dsum + 1e-16)


def _rdenom1(dparts):
    return pl.pallas_call(
        _rdenom1_body,
        out_shape=jax.ShapeDtypeStruct((1, N * HEADS), f32),
    )(dparts)


def _mid_body(p_ref, b1_ref, w2_ref, a2_ref, rd_ref, ex_ref,
              h2_ref, as2_ref, ad2_ref, sh2_ref):
    rdexp = jnp.dot(rd_ref[...], ex_ref[...], preferred_element_type=f32)
    o1 = (p_ref[0] + p_ref[1]) * rdexp + b1_ref[...]
    x2 = jnp.where(o1 > 0.0, o1, jnp.exp(jnp.minimum(o1, 0.0)) - 1.0)
    h2 = jnp.dot(x2, w2_ref[...], preferred_element_type=f32)
    h2_ref[...] = h2
    a2 = jnp.dot(h2, a2_ref[...], preferred_element_type=f32)
    a_s = a2[:, 0:1]
    a_d = a2[:, 1:2]
    as2_ref[...] = a_s
    ad2_ref[...] = a_d
    m = jnp.max(a_s) + jnp.max(a_d)
    sh2_ref[...] = _leaky(jnp.full((1, 16), m, dtype=f32))


def _mid(parts, bias1, W2p, A2, rd8, expand):
    return pl.pallas_call(
        _mid_body,
        out_shape=[
            jax.ShapeDtypeStruct((N, D2P), f32),
            jax.ShapeDtypeStruct((N, 1), f32),
            jax.ShapeDtypeStruct((N, 1), f32),
            jax.ShapeDtypeStruct((1, 16), f32),
        ],
    )(parts, bias1.reshape(1, D1), W2p, A2, rd8, expand)


def _rdenom2_body(d_ref, rd_ref):
    dsum = jnp.sum(d_ref[...], axis=0, keepdims=True)
    rd_ref[...] = 1.0 / (dsum + 1e-16)


def _rdenom2(dparts):
    return pl.pallas_call(
        _rdenom2_body,
        out_shape=jax.ShapeDtypeStruct((1, N), f32),
    )(dparts)


def _fin_body(p_ref, b2_ref, rd_ref, o_ref):
    o = (p_ref[0] + p_ref[1]) * rd_ref[...]
    o_ref[...] = o[:, :NC_OUT] + b2_ref[...]


def _fin(parts, bias2, rd2):
    return pl.pallas_call(
        _fin_body,
        out_shape=jax.ShapeDtypeStruct((N, NC_OUT), f32),
    )(parts, bias2.reshape(1, NC_OUT), rd2)


@functools.partial(
    pl.kernel,
    mesh=_MESH,
    compiler_params=pltpu.CompilerParams(needs_layout_passes=False, use_tc_tiling_on_sc=False),
    out_type=[
        jax.ShapeDtypeStruct((E * HEADS,), f32),
        jax.ShapeDtypeStruct((NW, N * HEADS), f32),
    ],
    scratch_types=[
        pltpu.VMEM((2, CH_A), i32),
        pltpu.VMEM((2, CH_A), i32),
        pltpu.VMEM((2, CH_A, 16), f32),
        pltpu.VMEM((2, CH_A, 16), f32),
        pltpu.VMEM((CH_A * HEADS,), f32),
        pltpu.VMEM((N * HEADS,), f32),
        pltpu.VMEM((16,), f32),
        pltpu.SemaphoreType.DMA,
        pltpu.SemaphoreType.DMA,
    ],
)
def _attn1(src_h, dst_h, as_h, ad_h, sh_h, ex_h, dp_h,
           srcb, dstb, g1, g2, exo, dacc, shv, sem0, sem1):
    wid = lax.axis_index("s") * NCORES + lax.axis_index("c")
    base = wid * PW
    sems = (sem0, sem1)
    pltpu.sync_copy(sh_h, shv)
    shift = shv[...]

    zero16 = jnp.zeros((16,), f32)

    def _z(i, c):
        dacc[pl.ds(i * 16, 16)] = zero16
        return c

    lax.fori_loop(0, N * HEADS // 16, _z, 0)

    def _start(c, b):
        off = base + c * CH_A
        pltpu.sync_copy(src_h.at[pl.ds(off, CH_A)], srcb.at[b])
        pltpu.sync_copy(dst_h.at[pl.ds(off, CH_A)], dstb.at[b])
        pltpu.async_copy(as_h.at[srcb.at[b]], g1.at[b], sems[b])
        pltpu.async_copy(ad_h.at[dstb.at[b]], g2.at[b], sems[b])

    def _finish(c, b):
        off = base + c * CH_A
        pltpu.make_async_copy(as_h.at[srcb.at[b]], g1.at[b], sems[b]).wait()
        pltpu.make_async_copy(ad_h.at[dstb.at[b]], g2.at[b], sems[b]).wait()

        def _group(g, cc):
            lane = jax.lax.broadcasted_iota(i32, (16,), 0)
            off8 = lane & 7
            hi = lane >> 3
            dvec = dstb[b, pl.ds(g * 16, 16)]
            for p in range(8):
                e0 = g * 16 + 2 * p
                e1 = e0 + 1
                a = jnp.where(lane < 8,
                              g1[b, e0, :] + g2[b, e0, :],
                              g1[b, e1, :] + g2[b, e1, :])
                ex = jnp.exp(_leaky(a) - shift)
                exo[pl.ds(g * 128 + p * 16, 16)] = ex
                sel = 2 * p + hi
                dd = dvec.at[sel].get(mode="promise_in_bounds")
                plsc.addupdate_scatter(dacc, [dd * HEADS + off8], ex)
            return cc

        lax.fori_loop(0, CH_A // 16, _group, 0)
        pltpu.sync_copy(exo, ex_h.at[pl.ds(off * HEADS, CH_A * HEADS)])

    _start(0, 0)

    def _outer(g, carry):
        c0 = g * 2
        _start(c0 + 1, 1)
        _finish(c0, 0)

        @pl.when(c0 + 2 < NCH_A)
        def _s():
            _start(c0 + 2, 0)

        _finish(c0 + 1, 1)
        return carry

    lax.fori_loop(0, NCH_A // 2, _outer, 0)
    if NCH_A % 2 == 1:
        _finish(NCH_A - 1, 0)
    pltpu.sync_copy(dacc, dp_h.at[wid])


@functools.partial(
    pl.kernel,
    mesh=_MESH,
    compiler_params=pltpu.CompilerParams(needs_layout_passes=False, use_tc_tiling_on_sc=False),
    out_type=jax.ShapeDtypeStruct((NCORES, N, D1), f32),
    scratch_types=[
        pltpu.VMEM((CH_B1,), i32),
        pltpu.VMEM((96,), i32),
        pltpu.VMEM((104,), i32),
        pltpu.VMEM((CH_B1, D1), f32),
        pltpu.VMEM((CH_B1 * HEADS,), f32),
        pltpu.VMEM_SHARED((N, D1), f32),
        pltpu.SemaphoreType.DMA,
        pltpu.SemaphoreType.DMA,
    ],
)
def _msg1(src_h, dst_h, h1_h, ex_h, out_h,
          srcb, dstbA, dstbB, hb, exb, osp, sem1, semS):
    cc = lax.axis_index("c")
    sid = lax.axis_index("s")
    wid = sid * NCORES + cc
    base = wid * PW

    zero16 = jnp.zeros((16,), f32)
    for j in range(8):
        def _zr(r, c, j=j):
            hb[r, pl.ds(j * 16, 16)] = zero16
            return c
        lax.fori_loop(0, 125, _zr, 0)
    for k in range(5):
        pltpu.sync_copy(hb.at[pl.ds(0, 125)],
                        osp.at[pl.ds(sid * ROWS_T + k * 125, 125)])
    plsc.subcore_barrier()

    def _pair(p, ccar):
        zl = jax.lax.broadcasted_iota(i32, (16,), 0) * 0
        cvec = exb[pl.ds(p * 16, 16)]
        for half in range(2):
            e = 2 * p + half
            for j in range(HEADS):
                cj = cvec.at[zl + half * 8 + j].get(mode="promise_in_bounds")
                hb[e, pl.ds(j * 16, 16)] = hb[e, pl.ds(j * 16, 16)] * cj
        return ccar

    def _chunk(c, carry):
        off = base + c * CH_B1

        @pl.when(c > 0)
        def _d():
            pltpu.make_async_copy(hb.at[pl.ds(0, 96)], osp.at[dstbA], semS).wait()

        pltpu.sync_copy(src_h.at[pl.ds(off, CH_B1)], srcb)
        pltpu.sync_copy(dst_h.at[pl.ds(off, 96)], dstbA)
        pltpu.sync_copy(dst_h.at[pl.ds(off + 96, 104)], dstbB)
        cp1 = pltpu.async_copy(h1_h.at[srcb], hb, sem1)
        pltpu.sync_copy(ex_h.at[pl.ds(off * HEADS, CH_B1 * HEADS)], exb)
        cp1.wait()

        lax.fori_loop(0, 48, _pair, 0)
        pltpu.async_copy(hb.at[pl.ds(0, 96)], osp.at[dstbA], semS, add=True)
        lax.fori_loop(48, CH_B1 // 2, _pair, 0)
        pltpu.sync_copy(hb.at[pl.ds(96, 104)], osp.at[dstbB], add=True)
        return carry

    lax.fori_loop(0, NCH_B1, _chunk, 0)
    pltpu.make_async_copy(hb.at[pl.ds(0, 96)], osp.at[dstbA], semS).wait()
    plsc.subcore_barrier()

    @pl.when(sid == 0)
    def _copy_out():
        pltpu.sync_copy(osp, out_h.at[cc])


@functools.partial(
    pl.kernel,
    mesh=_MESH,
    compiler_params=pltpu.CompilerParams(needs_layout_passes=False, use_tc_tiling_on_sc=False),
    out_type=[
        jax.ShapeDtypeStruct((E,), f32),
        jax.ShapeDtypeStruct((NW, N), f32),
    ],
    scratch_types=[
        pltpu.VMEM((2, CH_A2), i32),
        pltpu.VMEM((2, CH_A2), i32),
        pltpu.VMEM((2, CH_A2), f32),
        pltpu.VMEM((2, CH_A2), f32),
        pltpu.VMEM((N,), f32),
        pltpu.VMEM((16,), f32),
        pltpu.SemaphoreType.DMA,
        pltpu.SemaphoreType.DMA,
    ],
)
def _attn2(src_h, dst_h, as_h, ad_h, sh_h, ex_h, dp_h,
           srcb, dstb, gs, gd, dacc, shv, sem0, sem1):
    wid = lax.axis_index("s") * NCORES + lax.axis_index("c")
    base = wid * PW
    sems = (sem0, sem1)
    pltpu.sync_copy(sh_h, shv)
    shift = shv[...]

    zero16 = jnp.zeros((16,), f32)

    def _z(i, c):
        dacc[pl.ds(i * 16, 16)] = zero16
        return c

    lax.fori_loop(0, N // 16, _z, 0)

    def _start(c, b):
        off = base + c * CH_A2
        pltpu.sync_copy(src_h.at[pl.ds(off, CH_A2)], srcb.at[b])
        pltpu.sync_copy(dst_h.at[pl.ds(off, CH_A2)], dstb.at[b])
        pltpu.async_copy(as_h.at[srcb.at[b]], gs.at[b], sems[b])
        pltpu.async_copy(ad_h.at[dstb.at[b]], gd.at[b], sems[b])

    def _finish(c, b):
        off = base + c * CH_A2
        pltpu.make_async_copy(as_h.at[srcb.at[b]], gs.at[b], sems[b]).wait()
        pltpu.make_async_copy(ad_h.at[dstb.at[b]], gd.at[b], sems[b]).wait()

        def _group(g, cg):
            a = _leaky(gs[b, pl.ds(g * 16, 16)] + gd[b, pl.ds(g * 16, 16)])
            ex = jnp.exp(a - shift)
            gs[b, pl.ds(g * 16, 16)] = ex
            dvec = dstb[b, pl.ds(g * 16, 16)]
            plsc.addupdate_scatter(dacc, [dvec], ex)
            return cg

        lax.fori_loop(0, CH_A2 // 16, _group, 0)
        pltpu.sync_copy(gs.at[b], ex_h.at[pl.ds(off, CH_A2)])

    _start(0, 0)

    def _outer(g, carry):
        c0 = g * 2
        _start(c0 + 1, 1)
        _finish(c0, 0)

        @pl.when(c0 + 2 < NCH_A2)
        def _s():
            _start(c0 + 2, 0)

        _finish(c0 + 1, 1)
        return carry

    lax.fori_loop(0, NCH_A2 // 2, _outer, 0)
    if NCH_A2 % 2 == 1:
        _finish(NCH_A2 - 1, 0)
    pltpu.sync_copy(dacc, dp_h.at[wid])


@functools.partial(
    pl.kernel,
    mesh=_MESH,
    compiler_params=pltpu.CompilerParams(needs_layout_passes=False, use_tc_tiling_on_sc=False),
    out_type=jax.ShapeDtypeStruct((NCORES, N, D2P), f32),
    scratch_types=[
        pltpu.VMEM((2, CH_B2), i32),
        pltpu.VMEM((2, CH_B2), i32),
        pltpu.VMEM((2, CH_B2, D2P), f32),
        pltpu.VMEM((2, CH_B2), f32),
        pltpu.VMEM_SHARED((N, D2P), f32),
        pltpu.SemaphoreType.DMA,
        pltpu.SemaphoreType.DMA,
    ],
)
def _msg2(src_h, dst_h, h2_h, ex_h, out_h,
          srcb, dstb, hb, exb, osp, sem0, sem1):
    cc = lax.axis_index("c")
    sid = lax.axis_index("s")
    wid = sid * NCORES + cc
    base = wid * PW
    sems = (sem0, sem1)

    zero16 = jnp.zeros((16,), f32)
    for j in range(D2P // 16):
        def _zr(r, c, j=j):
            hb[0, r, pl.ds(j * 16, 16)] = zero16
            return c
        lax.fori_loop(0, 125, _zr, 0)
    for k in range(5):
        pltpu.sync_copy(hb.at[0, pl.ds(0, 125)],
                        osp.at[pl.ds(sid * ROWS_T + k * 125, 125)])
    plsc.subcore_barrier()

    def _start(c, b):
        off = base + c * CH_B2
        pltpu.sync_copy(src_h.at[pl.ds(off, CH_B2)], srcb.at[b])
        pltpu.sync_copy(dst_h.at[pl.ds(off, CH_B2)], dstb.at[b])
        pltpu.sync_copy(ex_h.at[pl.ds(off, CH_B2)], exb.at[b])
        pltpu.async_copy(h2_h.at[srcb.at[b]], hb.at[b], sems[b])

    def _finish(b):
        pltpu.make_async_copy(h2_h.at[srcb.at[b]], hb.at[b], sems[b]).wait()

        def _group(g, cg):
            zl = jax.lax.broadcasted_iota(i32, (16,), 0) * 0
            cvec = exb[b, pl.ds(g * 16, 16)]
            for k in range(16):
                e = g * 16 + k
                ck = cvec.at[zl + k].get(mode="promise_in_bounds")
                for j in range(D2P // 16):
                    hb[b, e, pl.ds(j * 16, 16)] = hb[b, e, pl.ds(j * 16, 16)] * ck
            return cg

        lax.fori_loop(0, CH_B2 // 16, _group, 0)
        pltpu.sync_copy(hb.at[b], osp.at[dstb.at[b]], add=True)

    _start(0, 0)

    def _outer(g, carry):
        c0 = g * 2
        _start(c0 + 1, 1)
        _finish(0)

        @pl.when(c0 + 2 < NCH_B2)
        def _s():
            _start(c0 + 2, 0)

        _finish(1)
        return carry

    lax.fori_loop(0, NCH_B2 // 2, _outer, 0)
    if NCH_B2 % 2 == 1:
        _finish(0)
    plsc.subcore_barrier()

    @pl.when(sid == 0)
    def _copy_out():
        pltpu.sync_copy(osp, out_h.at[cc])


def kernel(x, edge_index, W1, att_src1, att_dst1, bias1, W2, att_src2, att_dst2, bias2):
    src = edge_index[0]
    dst = edge_index[1]

    rows = jnp.arange(D1)
    cols = rows // C1
    Bs8 = jnp.zeros((D1, HEADS), f32).at[rows, cols].set(att_src1.reshape(D1))
    Bd8 = jnp.zeros((D1, HEADS), f32).at[rows, cols].set(att_dst1.reshape(D1))
    Bs16 = jnp.concatenate([Bs8, Bs8], axis=1)
    Bd16 = jnp.concatenate([Bd8, Bd8], axis=1)

    h1, as16, ad16, sh1 = _pre1(x, W1, Bs16, Bd16)

    ex1, dparts1 = _attn1(src, dst, as16, ad16, sh1.reshape(16))
    rd1 = _rdenom1(dparts1)
    rd8 = rd1.reshape(N, HEADS)
    expand = jnp.repeat(jnp.eye(HEADS, dtype=f32), C1, axis=1)

    out1_parts = _msg1(src, dst, h1, ex1)

    W2p = jnp.concatenate([W2, jnp.zeros((D1, D2P - NC_OUT), f32)], axis=1)
    a2s = jnp.concatenate([att_src2.reshape(NC_OUT, 1),
                           jnp.zeros((D2P - NC_OUT, 1), f32)], axis=0)
    a2d = jnp.concatenate([att_dst2.reshape(NC_OUT, 1),
                           jnp.zeros((D2P - NC_OUT, 1), f32)], axis=0)
    A2 = jnp.concatenate([a2s, a2d], axis=1)

    h2, as2, ad2, sh2 = _mid(out1_parts, bias1, W2p, A2, rd8, expand)

    ex2, dparts2 = _attn2(src, dst, as2.reshape(N), ad2.reshape(N), sh2.reshape(16))
    rd2 = _rdenom2(dparts2).reshape(N, 1)

    out2_parts = _msg2(src, dst, h2, ex2)

    return _fin(out2_parts, bias2, rd2)
